# baseline (speedup 1.0000x reference)
import jax
import jax.numpy as jnp
from jax import lax
from jax.experimental import pallas as pl
from jax.experimental.pallas import tpu as pltpu
from jax.experimental.pallas import tpu_sc as plsc

_B, _T, _H, _W, _LW, _OD = 16, 256, 48, 64, 101, 128
_NF = _B * _T
_NBINS = 512
_NC, _NS = 2, 16
_TH = _T // 2
_PAD = (_LW - 1) // 2


def _sc_hist_body(ft_hbm, out_hbm, r0, g0, b0, r1, g1, b1, hist, sem0, sem1):
    cid = lax.axis_index("c")
    sid = lax.axis_index("s")
    wid = sid * _NC + cid
    bb = wid // 2
    tbase = (wid % 2) * _TH

    iota = lax.iota(jnp.int32, 16)
    zero16 = jnp.zeros((16,), jnp.int32)
    ones = jnp.ones((16,), jnp.int32)

    def start(h, bufs, sem):
        for c, buf in enumerate(bufs):
            pltpu.async_copy(
                ft_hbm.at[bb, h, c, :, pl.ds(tbase, _TH)], buf, sem)

    def wait(h, bufs, sem):
        for c, buf in enumerate(bufs):
            pltpu.make_async_copy(
                ft_hbm.at[bb, h, c, :, pl.ds(tbase, _TH)], buf, sem).wait()

    start(0, (r0, g0, b0), sem0)

    def zbody(i, c):
        for k in range(_NBINS // 16):
            hist[i, pl.ds(k * 16, 16)] = zero16
        return c

    lax.fori_loop(0, _TH, zbody, 0)

    def do_slab(rb, gb, bvb):
        @plsc.parallel_loop(0, 512, unroll=8)
        def pbody(j):
            wrow = j >> 3
            tc = (j & 7) * 16
            trow = tc + iota
            r = rb[wrow, pl.ds(tc, 16)]
            g = gb[wrow, pl.ds(tc, 16)]
            bv = bvb[wrow, pl.ds(tc, 16)]
            bins = ((r >> 5) << 6) + ((g >> 5) << 3) + (bv >> 5)
            plsc.addupdate_scatter(hist, [trow, bins], ones)

    npairs = _H // 2

    def pair_body(p, c):
        h = 2 * p
        start(h + 1, (r1, g1, b1), sem1)
        wait(h, (r0, g0, b0), sem0)
        do_slab(r0, g0, b0)

        @pl.when(p < npairs - 1)
        def _start_next():
            start(h + 2, (r0, g0, b0), sem0)

        wait(h + 1, (r1, g1, b1), sem1)
        do_slab(r1, g1, b1)
        return c

    lax.fori_loop(0, npairs, pair_body, 0)
    pltpu.sync_copy(hist, out_hbm.at[pl.ds(bb * _T + tbase, _TH)])


def _make_sc_hist(interpret=False):
    return pl.kernel(
        _sc_hist_body,
        out_type=jax.ShapeDtypeStruct((_NF, _NBINS), jnp.int32),
        mesh=plsc.VectorSubcoreMesh(
            core_axis_name="c", subcore_axis_name="s",
            num_cores=_NC, num_subcores=_NS),
        scratch_types=[
            pltpu.VMEM((_W, _TH), jnp.int32),
            pltpu.VMEM((_W, _TH), jnp.int32),
            pltpu.VMEM((_W, _TH), jnp.int32),
            pltpu.VMEM((_W, _TH), jnp.int32),
            pltpu.VMEM((_W, _TH), jnp.int32),
            pltpu.VMEM((_W, _TH), jnp.int32),
            pltpu.VMEM((_TH, _NBINS), jnp.int32),
            pltpu.SemaphoreType.DMA,
            pltpu.SemaphoreType.DMA,
        ],
        compiler_params=pltpu.CompilerParams(needs_layout_passes=False),
        interpret=interpret,
    )


_WN = 384


def _tc_dense_body(counts_ref, wr_ref, b_ref, out_ref):
    x = counts_ref[0].astype(jnp.float32)
    ssq = jnp.sum(x * x, axis=1, keepdims=True)
    xn = x * lax.rsqrt(ssq)
    sims = lax.dot_general(
        xn, xn, (((1,), (1,)), ((), ())),
        preferred_element_type=jnp.float32,
        precision=lax.Precision.DEFAULT)
    zpad = jnp.zeros((_T, _PAD), jnp.float32)
    zpad_r = jnp.zeros((_T, _WN - _T - _PAD), jnp.float32)
    padded = jnp.concatenate([zpad, sims, zpad_r], axis=1)
    ir = lax.broadcasted_iota(jnp.int32, (_WN, _WN), 0)
    ic = lax.broadcasted_iota(jnp.int32, (_WN, _WN), 1)
    perm = jnp.where(ir + ic == _WN - 1, 1.0, 0.0)
    rev = lax.dot_general(
        padded, perm, (((1,), (0,)), ((), ())),
        preferred_element_type=jnp.float32,
        precision=lax.Precision.DEFAULT)
    r2 = pltpu.roll(rev, 0, 1, stride=1, stride_axis=0)
    y = lax.dot_general(
        r2, wr_ref[:, :], (((1,), (0,)), ((), ())),
        preferred_element_type=jnp.float32,
        precision=lax.Precision.DEFAULT)
    out_ref[0] = jnp.maximum(y + b_ref[0], 0.0)


def _tc_dense(counts3, wr, b2d, interpret=False):
    return pl.pallas_call(
        _tc_dense_body,
        grid=(_B,),
        in_specs=[
            pl.BlockSpec((1, _T, _NBINS), lambda i: (i, 0, 0)),
            pl.BlockSpec((_WN, _OD), lambda i: (0, 0)),
            pl.BlockSpec((1, _OD), lambda i: (0, 0)),
        ],
        out_specs=pl.BlockSpec((1, _T, _OD), lambda i: (i, 0, 0)),
        out_shape=jax.ShapeDtypeStruct((_B, _T, _OD), jnp.float32),
        interpret=interpret,
    )(counts3, wr, b2d)


def kernel(frames, W, b):
    ft = jnp.transpose(frames.astype(jnp.int32), (0, 2, 4, 3, 1))
    counts = _make_sc_hist()(ft)
    counts3 = counts.reshape(_B, _T, _NBINS)
    wr = jnp.concatenate(
        [jnp.zeros((_WN - _LW, _OD), jnp.float32), W[::-1, :]], axis=0)
    return _tc_dense(counts3, wr, b.reshape(1, _OD))


# speedup vs baseline: 1.0301x; 1.0301x over previous
import jax
import jax.numpy as jnp
from jax import lax
from jax.experimental import pallas as pl
from jax.experimental.pallas import tpu as pltpu
from jax.experimental.pallas import tpu_sc as plsc

_B, _T, _H, _W, _LW, _OD = 16, 256, 48, 64, 101, 128
_NF = _B * _T
_NBINS = 512
_NC, _NS = 2, 16
_TH = _T // 2
_PAD = (_LW - 1) // 2


def _sc_hist_body(ft_hbm, out_hbm, r0, g0, b0, r1, g1, b1, hist, sem0, sem1):
    cid = lax.axis_index("c")
    sid = lax.axis_index("s")
    wid = sid * _NC + cid
    bb = wid // 2
    tbase = (wid % 2) * _TH

    iota = lax.iota(jnp.int32, 16)
    zero16 = jnp.zeros((16,), jnp.int32)
    ones = jnp.ones((16,), jnp.int32)

    def start(h, bufs, sem):
        for c, buf in enumerate(bufs):
            pltpu.async_copy(
                ft_hbm.at[bb, h, c, :, pl.ds(tbase, _TH)], buf, sem)

    def wait(h, bufs, sem):
        for c, buf in enumerate(bufs):
            pltpu.make_async_copy(
                ft_hbm.at[bb, h, c, :, pl.ds(tbase, _TH)], buf, sem).wait()

    start(0, (r0, g0, b0), sem0)

    def zbody(i, c):
        for k in range(_NBINS // 16):
            hist[i, pl.ds(k * 16, 16)] = zero16
        return c

    lax.fori_loop(0, _TH, zbody, 0)

    def do_slab(rb, gb, bvb):
        @plsc.parallel_loop(0, 512, unroll=8)
        def pbody(j):
            wrow = j >> 3
            tc = (j & 7) * 16
            trow = tc + iota
            r = rb[wrow, pl.ds(tc, 16)]
            g = gb[wrow, pl.ds(tc, 16)]
            bv = bvb[wrow, pl.ds(tc, 16)]
            bins = ((r >> 5) << 6) + ((g >> 5) << 3) + (bv >> 5)
            plsc.addupdate_scatter(hist, [trow, bins], ones)

    npairs = _H // 2

    def pair_body(p, c):
        h = 2 * p
        start(h + 1, (r1, g1, b1), sem1)
        wait(h, (r0, g0, b0), sem0)
        do_slab(r0, g0, b0)

        @pl.when(p < npairs - 1)
        def _start_next():
            start(h + 2, (r0, g0, b0), sem0)

        wait(h + 1, (r1, g1, b1), sem1)
        do_slab(r1, g1, b1)
        return c

    lax.fori_loop(0, npairs, pair_body, 0)
    pltpu.sync_copy(hist, out_hbm.at[pl.ds(bb * _T + tbase, _TH)])


def _make_sc_hist(interpret=False):
    return pl.kernel(
        _sc_hist_body,
        out_type=jax.ShapeDtypeStruct((_NF, _NBINS), jnp.int32),
        mesh=plsc.VectorSubcoreMesh(
            core_axis_name="c", subcore_axis_name="s",
            num_cores=_NC, num_subcores=_NS),
        scratch_types=[
            pltpu.VMEM((_W, _TH), jnp.int32),
            pltpu.VMEM((_W, _TH), jnp.int32),
            pltpu.VMEM((_W, _TH), jnp.int32),
            pltpu.VMEM((_W, _TH), jnp.int32),
            pltpu.VMEM((_W, _TH), jnp.int32),
            pltpu.VMEM((_W, _TH), jnp.int32),
            pltpu.VMEM((_TH, _NBINS), jnp.int32),
            pltpu.SemaphoreType.DMA,
            pltpu.SemaphoreType.DMA,
        ],
        compiler_params=pltpu.CompilerParams(needs_layout_passes=False),
        interpret=interpret,
    )


_WN = 384
_CPS = 2


def _tc_dense_body(counts_ref, wr_ref, b_ref, out_ref):
    ir = lax.broadcasted_iota(jnp.int32, (_WN, _WN), 0)
    ic = lax.broadcasted_iota(jnp.int32, (_WN, _WN), 1)
    perm = jnp.where(ir + ic == _WN - 1, 1.0, 0.0)
    zpad = jnp.zeros((_T, _PAD), jnp.float32)
    zpad_r = jnp.zeros((_T, _WN - _T - _PAD), jnp.float32)
    for k in range(_CPS):
        x = counts_ref[k].astype(jnp.float32)
        ssq = jnp.sum(x * x, axis=1, keepdims=True)
        xn = x * lax.rsqrt(ssq)
        sims = lax.dot_general(
            xn, xn, (((1,), (1,)), ((), ())),
            preferred_element_type=jnp.float32,
            precision=lax.Precision.DEFAULT)
        padded = jnp.concatenate([zpad, sims, zpad_r], axis=1)
        rev = lax.dot_general(
            padded, perm, (((1,), (0,)), ((), ())),
            preferred_element_type=jnp.float32,
            precision=lax.Precision.DEFAULT)
        r2 = pltpu.roll(rev, 0, 1, stride=1, stride_axis=0)
        y = lax.dot_general(
            r2, wr_ref[:, :], (((1,), (0,)), ((), ())),
            preferred_element_type=jnp.float32,
            precision=lax.Precision.DEFAULT)
        out_ref[k] = jnp.maximum(y + b_ref[0], 0.0)


def _tc_dense(counts3, wr, b2d, interpret=False):
    return pl.pallas_call(
        _tc_dense_body,
        grid=(_B // _CPS,),
        in_specs=[
            pl.BlockSpec((_CPS, _T, _NBINS), lambda i: (i, 0, 0)),
            pl.BlockSpec((_WN, _OD), lambda i: (0, 0)),
            pl.BlockSpec((1, _OD), lambda i: (0, 0)),
        ],
        out_specs=pl.BlockSpec((_CPS, _T, _OD), lambda i: (i, 0, 0)),
        out_shape=jax.ShapeDtypeStruct((_B, _T, _OD), jnp.float32),
        interpret=interpret,
    )(counts3, wr, b2d)


def kernel(frames, W, b):
    ft = jnp.transpose(frames.astype(jnp.int32), (0, 2, 4, 3, 1))
    counts = _make_sc_hist()(ft)
    counts3 = counts.reshape(_B, _T, _NBINS)
    wr = jnp.concatenate(
        [jnp.zeros((_WN - _LW, _OD), jnp.float32), W[::-1, :]], axis=0)
    return _tc_dense(counts3, wr, b.reshape(1, _OD))


# speedup vs baseline: 1.0620x; 1.0310x over previous
import jax
import jax.numpy as jnp
from jax import lax
from jax.experimental import pallas as pl
from jax.experimental.pallas import tpu as pltpu
from jax.experimental.pallas import tpu_sc as plsc

_B, _T, _H, _W, _LW, _OD = 16, 256, 48, 64, 101, 128
_NF = _B * _T
_NBINS = 512
_NC, _NS = 2, 16
_TH = _T // 2
_PAD = (_LW - 1) // 2


def _sc_hist_body(ft_hbm, out_hbm, r0, g0, b0, r1, g1, b1, hist, sem0, sem1):
    cid = lax.axis_index("c")
    sid = lax.axis_index("s")
    wid = sid * _NC + cid
    bb = wid // 2
    tbase = (wid % 2) * _TH

    iota = lax.iota(jnp.int32, 16)
    zero16 = jnp.zeros((16,), jnp.int32)
    ones = jnp.ones((16,), jnp.int32)

    def start(h, bufs, sem):
        for c, buf in enumerate(bufs):
            pltpu.async_copy(
                ft_hbm.at[bb, h, c, :, pl.ds(tbase, _TH)], buf, sem)

    def wait(h, bufs, sem):
        for c, buf in enumerate(bufs):
            pltpu.make_async_copy(
                ft_hbm.at[bb, h, c, :, pl.ds(tbase, _TH)], buf, sem).wait()

    start(0, (r0, g0, b0), sem0)

    def zbody(i, c):
        for k in range(_TH // 16):
            hist[i, pl.ds(k * 16, 16)] = zero16
        return c

    lax.fori_loop(0, _NBINS, zbody, 0)

    def do_slab(rb, gb, bvb):
        @plsc.parallel_loop(0, 512, unroll=8)
        def pbody(j):
            wrow = j >> 3
            tc = (j & 7) * 16
            trow = tc + iota
            r = rb[wrow, pl.ds(tc, 16)]
            g = gb[wrow, pl.ds(tc, 16)]
            bv = bvb[wrow, pl.ds(tc, 16)]
            bins = ((r >> 5) << 6) + ((g >> 5) << 3) + (bv >> 5)
            plsc.addupdate_scatter(hist, [bins, trow], ones)

    npairs = _H // 2

    def pair_body(p, c):
        h = 2 * p
        start(h + 1, (r1, g1, b1), sem1)
        wait(h, (r0, g0, b0), sem0)
        do_slab(r0, g0, b0)

        @pl.when(p < npairs - 1)
        def _start_next():
            start(h + 2, (r0, g0, b0), sem0)

        wait(h + 1, (r1, g1, b1), sem1)
        do_slab(r1, g1, b1)
        return c

    lax.fori_loop(0, npairs, pair_body, 0)
    pltpu.sync_copy(hist, out_hbm.at[:, pl.ds(bb * _T + tbase, _TH)])


def _make_sc_hist(interpret=False):
    return pl.kernel(
        _sc_hist_body,
        out_type=jax.ShapeDtypeStruct((_NBINS, _NF), jnp.int32),
        mesh=plsc.VectorSubcoreMesh(
            core_axis_name="c", subcore_axis_name="s",
            num_cores=_NC, num_subcores=_NS),
        scratch_types=[
            pltpu.VMEM((_W, _TH), jnp.int32),
            pltpu.VMEM((_W, _TH), jnp.int32),
            pltpu.VMEM((_W, _TH), jnp.int32),
            pltpu.VMEM((_W, _TH), jnp.int32),
            pltpu.VMEM((_W, _TH), jnp.int32),
            pltpu.VMEM((_W, _TH), jnp.int32),
            pltpu.VMEM((_NBINS, _TH), jnp.int32),
            pltpu.SemaphoreType.DMA,
            pltpu.SemaphoreType.DMA,
        ],
        compiler_params=pltpu.CompilerParams(needs_layout_passes=False),
        interpret=interpret,
    )


_WN = 384
_CPS = 2


def _tc_dense_body(counts_ref, wr_ref, b_ref, out_ref):
    ir = lax.broadcasted_iota(jnp.int32, (_WN, _WN), 0)
    ic = lax.broadcasted_iota(jnp.int32, (_WN, _WN), 1)
    perm = jnp.where(ir + ic == _WN - 1, 1.0, 0.0)
    zpad = jnp.zeros((_T, _PAD), jnp.float32)
    zpad_r = jnp.zeros((_T, _WN - _T - _PAD), jnp.float32)
    for k in range(_CPS):
        x = counts_ref[:, pl.ds(k * _T, _T)].astype(jnp.float32)
        ssq = jnp.sum(x * x, axis=0, keepdims=True)
        xn = x * lax.rsqrt(ssq)
        sims = lax.dot_general(
            xn, xn, (((0,), (0,)), ((), ())),
            preferred_element_type=jnp.float32,
            precision=lax.Precision.DEFAULT)
        padded = jnp.concatenate([zpad, sims, zpad_r], axis=1)
        rev = lax.dot_general(
            padded, perm, (((1,), (0,)), ((), ())),
            preferred_element_type=jnp.float32,
            precision=lax.Precision.DEFAULT)
        r2 = pltpu.roll(rev, 0, 1, stride=1, stride_axis=0)
        y = lax.dot_general(
            r2, wr_ref[:, :], (((1,), (0,)), ((), ())),
            preferred_element_type=jnp.float32,
            precision=lax.Precision.DEFAULT)
        out_ref[k] = jnp.maximum(y + b_ref[0], 0.0)


def _tc_dense(counts3, wr, b2d, interpret=False):
    return pl.pallas_call(
        _tc_dense_body,
        grid=(_B // _CPS,),
        in_specs=[
            pl.BlockSpec((_NBINS, _CPS * _T), lambda i: (0, i)),
            pl.BlockSpec((_WN, _OD), lambda i: (0, 0)),
            pl.BlockSpec((1, _OD), lambda i: (0, 0)),
        ],
        out_specs=pl.BlockSpec((_CPS, _T, _OD), lambda i: (i, 0, 0)),
        out_shape=jax.ShapeDtypeStruct((_B, _T, _OD), jnp.float32),
        interpret=interpret,
    )(counts3, wr, b2d)


def kernel(frames, W, b):
    ft = jnp.transpose(frames.astype(jnp.int32), (0, 2, 4, 3, 1))
    counts = _make_sc_hist()(ft)
    wr = jnp.concatenate(
        [jnp.zeros((_WN - _LW, _OD), jnp.float32), W[::-1, :]], axis=0)
    return _tc_dense(counts, wr, b.reshape(1, _OD))
